# baseline (speedup 1.0000x reference)
import jax
import jax.numpy as jnp
from jax.experimental import pallas as pl
from jax.experimental.pallas import tpu as pltpu

N = 100000
D_IN = 128
D_HID = 256
D_OUT = 128
BLOCK = 10000
NBUF = 3


def _outer(x_hbm, w1_ref, b1_ref, w2_ref, b2_ref, out_hbm):
    w1b = w1_ref[...].astype(jnp.bfloat16)
    w2b = w2_ref[...].astype(jnp.bfloat16)
    b1v = b1_ref[...]
    b2v = b2_ref[...]

    def inner(idxs, x_ref, out_ref):
        i = idxs[0]
        half = jax.lax.rem(i, 2)
        xb = x_ref[pl.ds(half * (BLOCK // 2), BLOCK // 2), :].astype(jnp.bfloat16)
        h = jnp.dot(xb, w1b, preferred_element_type=jnp.float32)
        h = jnp.maximum(h + b1v, 0.0).astype(jnp.bfloat16)
        out = jnp.dot(h, w2b, preferred_element_type=jnp.float32)
        out_ref[...] = out + b2v

    pltpu.emit_pipeline(
        inner,
        grid=(2 * N // BLOCK,),
        in_specs=[
            pl.BlockSpec(
                (BLOCK, D_IN), lambda i: (i // 2, 0),
                pipeline_mode=pl.Buffered(buffer_count=NBUF),
            )
        ],
        out_specs=[pl.BlockSpec((BLOCK // 2, D_OUT), lambda i: (i, 0))],
        _explicit_indices=True,
    )(x_hbm, out_hbm)


@jax.jit
def kernel(x, W1, b1, W2, b2):
    b1r = b1.reshape(1, D_HID)
    b2r = b2.reshape(1, D_OUT)
    return pl.pallas_call(
        _outer,
        in_specs=[
            pl.BlockSpec(memory_space=pltpu.MemorySpace.HBM),
            pl.BlockSpec(memory_space=pltpu.MemorySpace.VMEM),
            pl.BlockSpec(memory_space=pltpu.MemorySpace.VMEM),
            pl.BlockSpec(memory_space=pltpu.MemorySpace.VMEM),
            pl.BlockSpec(memory_space=pltpu.MemorySpace.VMEM),
        ],
        out_specs=pl.BlockSpec(memory_space=pltpu.MemorySpace.HBM),
        out_shape=jax.ShapeDtypeStruct((N, D_OUT), jnp.float32),
    )(x, W1, b1r, W2, b2r)


# speedup vs baseline: 1.1657x; 1.1657x over previous
import jax
import jax.numpy as jnp
from jax.experimental import pallas as pl
from jax.experimental.pallas import tpu as pltpu

N = 100000
D_IN = 128
D_HID = 256
D_OUT = 128
BLOCK = 10000
NBUF = 4
K_OUT = 4
STEPS = N // BLOCK


def _outer(x_hbm, w1_ref, b1_ref, w2_ref, b2_ref, out_hbm, obuf, osem):
    w1b = w1_ref[...].astype(jnp.bfloat16)
    w2b = w2_ref[...].astype(jnp.bfloat16)
    b1v = b1_ref[...]
    b2v = b2_ref[...]

    def _out_copy(step, slot):
        return pltpu.make_async_copy(
            obuf.at[slot],
            out_hbm.at[pl.ds(step * BLOCK, BLOCK), :],
            osem.at[slot],
        )

    def inner(idxs, x_ref):
        i = idxs[0]
        slot = jax.lax.rem(i, K_OUT)

        @pl.when(i >= K_OUT)
        def _wait_prev():
            _out_copy(i - K_OUT, slot).wait()

        xb = x_ref[...].astype(jnp.bfloat16)
        h = jnp.dot(xb, w1b, preferred_element_type=jnp.float32)
        h = jnp.maximum(h + b1v, 0.0).astype(jnp.bfloat16)
        out = jnp.dot(h, w2b, preferred_element_type=jnp.float32)
        obuf[slot] = out + b2v
        _out_copy(i, slot).start()

    pltpu.emit_pipeline(
        inner,
        grid=(STEPS,),
        in_specs=[
            pl.BlockSpec(
                (BLOCK, D_IN), lambda i: (i, 0),
                pipeline_mode=pl.Buffered(buffer_count=NBUF),
            )
        ],
        out_specs=[],
        _explicit_indices=True,
    )(x_hbm)

    for j in range(max(0, STEPS - K_OUT), STEPS):
        _out_copy(j, j % K_OUT).wait()


@jax.jit
def kernel(x, W1, b1, W2, b2):
    b1r = b1.reshape(1, D_HID)
    b2r = b2.reshape(1, D_OUT)
    return pl.pallas_call(
        _outer,
        in_specs=[
            pl.BlockSpec(memory_space=pltpu.MemorySpace.HBM),
            pl.BlockSpec(memory_space=pltpu.MemorySpace.VMEM),
            pl.BlockSpec(memory_space=pltpu.MemorySpace.VMEM),
            pl.BlockSpec(memory_space=pltpu.MemorySpace.VMEM),
            pl.BlockSpec(memory_space=pltpu.MemorySpace.VMEM),
        ],
        out_specs=pl.BlockSpec(memory_space=pltpu.MemorySpace.HBM),
        out_shape=jax.ShapeDtypeStruct((N, D_OUT), jnp.float32),
        scratch_shapes=[
            pltpu.VMEM((K_OUT, BLOCK, D_OUT), jnp.float32),
            pltpu.SemaphoreType.DMA((K_OUT,)),
        ],
    )(x, W1, b1r, W2, b2r)
